# baseline (speedup 1.0000x reference)
import jax
import jax.numpy as jnp
from jax.experimental import pallas as pl


def _vq_body(z_ref, cb_ref, o_ref):
    zb = z_ref[0]
    cb = cb_ref[...]
    csqr = jnp.sum(cb * cb, axis=1, keepdims=True)
    xsqr = jnp.sum(zb * zb, axis=0, keepdims=True)
    m = jax.lax.dot_general(cb.astype(jnp.bfloat16), zb.astype(jnp.bfloat16),
                            (((1,), (0,)), ((), ())),
                            preferred_element_type=jnp.float32)
    dist = (csqr + xsqr) - 2.0 * m
    mn = jnp.min(dist, axis=0, keepdims=True)
    ids = jax.lax.broadcasted_iota(jnp.int32, dist.shape, 0)
    k = dist.shape[0]
    idx = jnp.min(jnp.where(dist == mn, ids, k), axis=0)
    o_ref[0, 0, :] = idx.astype(jnp.int32)


def kernel(z_e_x, codebook):
    b, d, h, w = z_e_x.shape
    t = h * w
    k = codebook.shape[0]
    z = z_e_x.reshape(b, d, t)
    out = pl.pallas_call(
        _vq_body,
        grid=(b,),
        in_specs=[
            pl.BlockSpec((1, d, t), lambda i: (i, 0, 0)),
            pl.BlockSpec((k, d), lambda i: (0, 0)),
        ],
        out_specs=pl.BlockSpec((1, 1, t), lambda i: (i, 0, 0)),
        out_shape=jax.ShapeDtypeStruct((b, 1, t), jnp.int32),
    )(z, codebook)
    return out.reshape(b, h, w)


# speedup vs baseline: 1.0791x; 1.0791x over previous
import jax
import jax.numpy as jnp
from jax.experimental import pallas as pl


def _vq_body(z_ref, cb_ref, o_ref):
    zb = z_ref[0]
    cb = cb_ref[...]
    csqr = jnp.sum(cb * cb, axis=1, keepdims=True)
    xsqr = jnp.sum(zb * zb, axis=0, keepdims=True)
    m2 = jax.lax.dot_general((-2.0 * cb).astype(jnp.bfloat16),
                             zb.astype(jnp.bfloat16),
                             (((1,), (0,)), ((), ())),
                             preferred_element_type=jnp.float32)
    dist = (csqr + xsqr) + m2
    mn = jnp.min(dist, axis=0, keepdims=True)
    k = dist.shape[0]
    ids = jax.lax.broadcasted_iota(jnp.int32, (k, 1), 0).astype(jnp.float32)
    idx = jnp.min(jnp.where(dist == mn, ids, jnp.float32(k)), axis=0)
    o_ref[0, 0, :] = idx.astype(jnp.int32)


def kernel(z_e_x, codebook):
    b, d, h, w = z_e_x.shape
    t = h * w
    k = codebook.shape[0]
    z = z_e_x.reshape(b, d, t)
    out = pl.pallas_call(
        _vq_body,
        grid=(b,),
        in_specs=[
            pl.BlockSpec((1, d, t), lambda i: (i, 0, 0)),
            pl.BlockSpec((k, d), lambda i: (0, 0)),
        ],
        out_specs=pl.BlockSpec((1, 1, t), lambda i: (i, 0, 0)),
        out_shape=jax.ShapeDtypeStruct((b, 1, t), jnp.int32),
    )(z, codebook)
    return out.reshape(b, h, w)


# speedup vs baseline: 1.0894x; 1.0096x over previous
import jax
import jax.numpy as jnp
from jax.experimental import pallas as pl


_BB = 4


def _vq_body(z_ref, cb_ref, o_ref):
    cb = cb_ref[...]
    k = cb.shape[0]
    csqr = jnp.sum(cb * cb, axis=1, keepdims=True)
    cbm2 = (-2.0 * cb).astype(jnp.bfloat16)
    ids = jax.lax.broadcasted_iota(jnp.int32, (k, 1), 0).astype(jnp.float32)
    for b in range(_BB):
        zb = z_ref[b]
        xsqr = jnp.sum(zb * zb, axis=0, keepdims=True)
        m2 = jax.lax.dot_general(cbm2, zb.astype(jnp.bfloat16),
                                 (((1,), (0,)), ((), ())),
                                 preferred_element_type=jnp.float32)
        dist = (csqr + xsqr) + m2
        mn = jnp.min(dist, axis=0, keepdims=True)
        idx = jnp.min(jnp.where(dist == mn, ids, jnp.float32(k)), axis=0)
        o_ref[b, 0, :] = idx.astype(jnp.int32)


def kernel(z_e_x, codebook):
    b, d, h, w = z_e_x.shape
    t = h * w
    k = codebook.shape[0]
    z = z_e_x.reshape(b, d, t)
    out = pl.pallas_call(
        _vq_body,
        grid=(b // _BB,),
        in_specs=[
            pl.BlockSpec((_BB, d, t), lambda i: (i, 0, 0)),
            pl.BlockSpec((k, d), lambda i: (0, 0)),
        ],
        out_specs=pl.BlockSpec((_BB, 1, t), lambda i: (i, 0, 0)),
        out_shape=jax.ShapeDtypeStruct((b, 1, t), jnp.int32),
    )(z, codebook)
    return out.reshape(b, h, w)


# speedup vs baseline: 1.2336x; 1.1324x over previous
import jax
import jax.numpy as jnp
from jax.experimental import pallas as pl

_BB = 4


def _vq_body(z_ref, cb_ref, o_ref):
    cb = cb_ref[...]
    k = cb.shape[0]
    nchunk = k // 8
    csqr = jnp.sum(cb * cb, axis=1, keepdims=True)
    cbm2 = (-2.0 * cb).astype(jnp.bfloat16)
    srow = jax.lax.broadcasted_iota(jnp.int32, (8, 1), 0).astype(jnp.float32)
    for b in range(_BB):
        zb = z_ref[b]
        t = zb.shape[1]
        xsqr = jnp.sum(zb * zb, axis=0, keepdims=True)
        m2 = jax.lax.dot_general(cbm2, zb.astype(jnp.bfloat16),
                                 (((1,), (0,)), ((), ())),
                                 preferred_element_type=jnp.float32)
        acc_val = (csqr[0:8] + xsqr) + m2[0:8]
        acc_c = jnp.zeros((8, t), jnp.float32)
        for c in range(1, nchunk):
            d = (csqr[8 * c:8 * c + 8] + xsqr) + m2[8 * c:8 * c + 8]
            better = d < acc_val
            acc_val = jnp.where(better, d, acc_val)
            acc_c = jnp.where(better, jnp.float32(c), acc_c)
        ks = acc_c * 8.0 + srow
        mn8 = jnp.min(acc_val, axis=0, keepdims=True)
        idx = jnp.min(jnp.where(acc_val == mn8, ks, jnp.float32(k)), axis=0)
        o_ref[b, 0, :] = idx.astype(jnp.int32)


def kernel(z_e_x, codebook):
    b, d, h, w = z_e_x.shape
    t = h * w
    k = codebook.shape[0]
    z = z_e_x.reshape(b, d, t)
    out = pl.pallas_call(
        _vq_body,
        grid=(b // _BB,),
        in_specs=[
            pl.BlockSpec((_BB, d, t), lambda i: (i, 0, 0)),
            pl.BlockSpec((k, d), lambda i: (0, 0)),
        ],
        out_specs=pl.BlockSpec((_BB, 1, t), lambda i: (i, 0, 0)),
        out_shape=jax.ShapeDtypeStruct((b, 1, t), jnp.int32),
    )(z, codebook)
    return out.reshape(b, h, w)
